# baseline (speedup 1.0000x reference)
import jax
import jax.numpy as jnp
from jax import lax
from jax.experimental import pallas as pl
from jax.experimental.pallas import tpu as pltpu
from jax.experimental.pallas import tpu_sc as plsc

_N = 8192
_D = 2048
_NC = 2
_NS = 16
_NW = _NC * _NS
_ROWS_PER_W = _N // _NW
_R = 8
_CW = _R * _D
_CHUNKS = _ROWS_PER_W // _R
_PAIRS = _CHUNKS // 2
_L = 16
_JG = _D // _L


def _permute_chunk(idx_v, src, dst):
    @plsc.parallel_loop(0, _JG, 1, unroll=4)
    def _(jg):
        col = idx_v[pl.ds(jg * _L, _L)]
        for r in range(_R):
            vals = plsc.load_gather(src.at[pl.ds(r * _D, _D)], [col])
            dst[pl.ds(r * _D + jg * _L, _L)] = vals


def _sc_body(x_hbm, idx_hbm, out_hbm,
             idx_v, in0, in1, out0, out1,
             sem_i0, sem_i1, sem_o0, sem_o1):
    wid = lax.axis_index("s") * _NC + lax.axis_index("c")
    base = wid * _ROWS_PER_W * _D
    pltpu.sync_copy(idx_hbm, idx_v)

    pltpu.async_copy(x_hbm.at[pl.ds(base, _CW)], in0, sem_i0)
    pltpu.async_copy(x_hbm.at[pl.ds(base + _CW, _CW)], in1, sem_i1)

    def pair_body(i, carry):
        e0 = base + (2 * i) * _CW
        e1 = e0 + _CW

        pltpu.make_async_copy(x_hbm.at[pl.ds(e0, _CW)], in0, sem_i0).wait()

        @pl.when(i > 0)
        def _():
            pltpu.make_async_copy(out0, out_hbm.at[pl.ds(e0 - 2 * _CW, _CW)],
                                  sem_o0).wait()

        _permute_chunk(idx_v, in0, out0)
        pltpu.async_copy(out0, out_hbm.at[pl.ds(e0, _CW)], sem_o0)

        @pl.when(i < _PAIRS - 1)
        def _():
            pltpu.async_copy(x_hbm.at[pl.ds(e0 + 2 * _CW, _CW)], in0, sem_i0)

        pltpu.make_async_copy(x_hbm.at[pl.ds(e1, _CW)], in1, sem_i1).wait()

        @pl.when(i > 0)
        def _():
            pltpu.make_async_copy(out1, out_hbm.at[pl.ds(e1 - 2 * _CW, _CW)],
                                  sem_o1).wait()

        _permute_chunk(idx_v, in1, out1)
        pltpu.async_copy(out1, out_hbm.at[pl.ds(e1, _CW)], sem_o1)

        @pl.when(i < _PAIRS - 1)
        def _():
            pltpu.async_copy(x_hbm.at[pl.ds(e1 + 2 * _CW, _CW)], in1, sem_i1)

        return carry

    lax.fori_loop(0, _PAIRS, pair_body, 0)

    last0 = base + (_CHUNKS - 2) * _CW
    last1 = base + (_CHUNKS - 1) * _CW
    pltpu.make_async_copy(out0, out_hbm.at[pl.ds(last0, _CW)], sem_o0).wait()
    pltpu.make_async_copy(out1, out_hbm.at[pl.ds(last1, _CW)], sem_o1).wait()


@jax.jit
def kernel(inputs, idxs):
    mesh = plsc.VectorSubcoreMesh(
        core_axis_name="c", subcore_axis_name="s",
        num_cores=_NC, num_subcores=_NS,
    )
    f = pl.kernel(
        _sc_body,
        out_type=jax.ShapeDtypeStruct((_N * _D,), jnp.float32),
        mesh=mesh,
        scratch_types=[
            pltpu.VMEM((_D,), jnp.int32),
            pltpu.VMEM((_CW,), jnp.float32),
            pltpu.VMEM((_CW,), jnp.float32),
            pltpu.VMEM((_CW,), jnp.float32),
            pltpu.VMEM((_CW,), jnp.float32),
            pltpu.SemaphoreType.DMA,
            pltpu.SemaphoreType.DMA,
            pltpu.SemaphoreType.DMA,
            pltpu.SemaphoreType.DMA,
        ],
        compiler_params=pltpu.CompilerParams(needs_layout_passes=False),
    )
    return f(inputs.reshape(_N * _D), idxs).reshape(_N, _D)


# speedup vs baseline: 2.7133x; 2.7133x over previous
import jax
import jax.numpy as jnp
from jax import lax
from jax.experimental import pallas as pl
from jax.experimental.pallas import tpu as pltpu
from jax.experimental.pallas import tpu_sc as plsc

_N = 8192
_D = 2048
_NC = 2
_NS = 16
_NW = _NC * _NS
_ROWS_PER_W = _N // _NW
_R = 8
_CW = _R * _D
_CHUNKS = _ROWS_PER_W // _R
_PAIRS = _CHUNKS // 2
_L = 16
_JG = _D // _L


def _permute_chunk(idx_v, src, dst):
    @plsc.parallel_loop(0, _JG, 1, unroll=8)
    def _(jg):
        col = idx_v[pl.ds(jg * _L, _L)]
        for r in range(_R):
            row_i = jnp.full((_L,), r, dtype=jnp.int32)
            dst[r, pl.ds(jg * _L, _L)] = plsc.load_gather(src, [row_i, col])


def _sc_body(x_hbm, idx_hbm, out_hbm,
             idx_v, in0, in1, out0, out1,
             sem_i0, sem_i1, sem_o0, sem_o1):
    wid = lax.axis_index("s") * _NC + lax.axis_index("c")
    base = wid * _ROWS_PER_W
    pltpu.sync_copy(idx_hbm, idx_v)

    pltpu.async_copy(x_hbm.at[pl.ds(base, _R)], in0, sem_i0)
    pltpu.async_copy(x_hbm.at[pl.ds(base + _R, _R)], in1, sem_i1)

    def pair_body(i, carry):
        r0 = base + (2 * i) * _R
        r1 = r0 + _R

        pltpu.make_async_copy(x_hbm.at[pl.ds(r0, _R)], in0, sem_i0).wait()

        @pl.when(i > 0)
        def _():
            pltpu.make_async_copy(out0, out_hbm.at[pl.ds(r0 - 2 * _R, _R)],
                                  sem_o0).wait()

        _permute_chunk(idx_v, in0, out0)
        pltpu.async_copy(out0, out_hbm.at[pl.ds(r0, _R)], sem_o0)

        @pl.when(i < _PAIRS - 1)
        def _():
            pltpu.async_copy(x_hbm.at[pl.ds(r0 + 2 * _R, _R)], in0, sem_i0)

        pltpu.make_async_copy(x_hbm.at[pl.ds(r1, _R)], in1, sem_i1).wait()

        @pl.when(i > 0)
        def _():
            pltpu.make_async_copy(out1, out_hbm.at[pl.ds(r1 - 2 * _R, _R)],
                                  sem_o1).wait()

        _permute_chunk(idx_v, in1, out1)
        pltpu.async_copy(out1, out_hbm.at[pl.ds(r1, _R)], sem_o1)

        @pl.when(i < _PAIRS - 1)
        def _():
            pltpu.async_copy(x_hbm.at[pl.ds(r1 + 2 * _R, _R)], in1, sem_i1)

        return carry

    lax.fori_loop(0, _PAIRS, pair_body, 0)

    last0 = base + (_CHUNKS - 2) * _R
    last1 = base + (_CHUNKS - 1) * _R
    pltpu.make_async_copy(out0, out_hbm.at[pl.ds(last0, _R)], sem_o0).wait()
    pltpu.make_async_copy(out1, out_hbm.at[pl.ds(last1, _R)], sem_o1).wait()


@jax.jit
def kernel(inputs, idxs):
    mesh = plsc.VectorSubcoreMesh(
        core_axis_name="c", subcore_axis_name="s",
        num_cores=_NC, num_subcores=_NS,
    )
    f = pl.kernel(
        _sc_body,
        out_type=jax.ShapeDtypeStruct((_N, _D), jnp.float32),
        mesh=mesh,
        scratch_types=[
            pltpu.VMEM((_D,), jnp.int32),
            pltpu.VMEM((_R, _D), jnp.float32),
            pltpu.VMEM((_R, _D), jnp.float32),
            pltpu.VMEM((_R, _D), jnp.float32),
            pltpu.VMEM((_R, _D), jnp.float32),
            pltpu.SemaphoreType.DMA,
            pltpu.SemaphoreType.DMA,
            pltpu.SemaphoreType.DMA,
            pltpu.SemaphoreType.DMA,
        ],
        compiler_params=pltpu.CompilerParams(needs_layout_passes=False),
    )
    return f(inputs, idxs)


# speedup vs baseline: 2.7449x; 1.0116x over previous
import jax
import jax.numpy as jnp
from jax import lax
from jax.experimental import pallas as pl
from jax.experimental.pallas import tpu as pltpu
from jax.experimental.pallas import tpu_sc as plsc

_N = 8192
_D = 2048
_NC = 2
_NS = 16
_NW = _NC * _NS
_ROWS_PER_W = _N // _NW
_R = 8
_CHUNKS = _ROWS_PER_W // _R
_NBUF = 3
_MAIN = _CHUNKS // _NBUF
_TAIL = _CHUNKS - _MAIN * _NBUF
_L = 16
_JG = _D // _L


def _permute_chunk(idx_v, src, dst):
    @plsc.parallel_loop(0, _JG, 1, unroll=8)
    def _(jg):
        col = idx_v[pl.ds(jg * _L, _L)]
        for r in range(_R):
            row_i = jnp.full((_L,), r, dtype=jnp.int32)
            dst[r, pl.ds(jg * _L, _L)] = plsc.load_gather(src, [row_i, col])


def _sc_body(x_hbm, idx_hbm, out_hbm, idx_v,
             in0, in1, in2, out0, out1, out2,
             si0, si1, si2, so0, so1, so2):
    wid = lax.axis_index("s") * _NC + lax.axis_index("c")
    base = wid * _ROWS_PER_W
    pltpu.sync_copy(idx_hbm, idx_v)

    ins = (in0, in1, in2)
    outs = (out0, out1, out2)
    sis = (si0, si1, si2)
    sos = (so0, so1, so2)

    for k in range(_NBUF):
        pltpu.async_copy(x_hbm.at[pl.ds(base + k * _R, _R)], ins[k], sis[k])

    def ring_body(i, carry):
        for k in range(_NBUF):
            row = base + (i * _NBUF + k) * _R
            pltpu.make_async_copy(x_hbm.at[pl.ds(row, _R)], ins[k],
                                  sis[k]).wait()

            @pl.when(i > 0)
            def _(k=k, row=row):
                pltpu.make_async_copy(outs[k],
                                      out_hbm.at[pl.ds(row - _NBUF * _R, _R)],
                                      sos[k]).wait()

            _permute_chunk(idx_v, ins[k], outs[k])
            pltpu.async_copy(outs[k], out_hbm.at[pl.ds(row, _R)], sos[k])

            if k < _TAIL:
                pltpu.async_copy(x_hbm.at[pl.ds(row + _NBUF * _R, _R)],
                                 ins[k], sis[k])
            else:
                @pl.when(i < _MAIN - 1)
                def _(k=k, row=row):
                    pltpu.async_copy(x_hbm.at[pl.ds(row + _NBUF * _R, _R)],
                                     ins[k], sis[k])

        return carry

    lax.fori_loop(0, _MAIN, ring_body, 0)

    for k in range(_TAIL):
        row = base + (_MAIN * _NBUF + k) * _R
        pltpu.make_async_copy(x_hbm.at[pl.ds(row, _R)], ins[k], sis[k]).wait()
        pltpu.make_async_copy(outs[k], out_hbm.at[pl.ds(row - _NBUF * _R, _R)],
                              sos[k]).wait()
        _permute_chunk(idx_v, ins[k], outs[k])
        pltpu.async_copy(outs[k], out_hbm.at[pl.ds(row, _R)], sos[k])

    for k in range(_NBUF):
        c = _MAIN * _NBUF + k if k < _TAIL else (_MAIN - 1) * _NBUF + k
        pltpu.make_async_copy(outs[k], out_hbm.at[pl.ds(base + c * _R, _R)],
                              sos[k]).wait()


@jax.jit
def kernel(inputs, idxs):
    mesh = plsc.VectorSubcoreMesh(
        core_axis_name="c", subcore_axis_name="s",
        num_cores=_NC, num_subcores=_NS,
    )
    f = pl.kernel(
        _sc_body,
        out_type=jax.ShapeDtypeStruct((_N, _D), jnp.float32),
        mesh=mesh,
        scratch_types=[
            pltpu.VMEM((_D,), jnp.int32),
            pltpu.VMEM((_R, _D), jnp.float32),
            pltpu.VMEM((_R, _D), jnp.float32),
            pltpu.VMEM((_R, _D), jnp.float32),
            pltpu.VMEM((_R, _D), jnp.float32),
            pltpu.VMEM((_R, _D), jnp.float32),
            pltpu.VMEM((_R, _D), jnp.float32),
            pltpu.SemaphoreType.DMA,
            pltpu.SemaphoreType.DMA,
            pltpu.SemaphoreType.DMA,
            pltpu.SemaphoreType.DMA,
            pltpu.SemaphoreType.DMA,
            pltpu.SemaphoreType.DMA,
        ],
        compiler_params=pltpu.CompilerParams(needs_layout_passes=False),
    )
    return f(inputs, idxs)
